# baseline (speedup 1.0000x reference)
import jax
import jax.numpy as jnp
from jax import lax
from jax.experimental import pallas as pl
from jax.experimental.pallas import tpu as pltpu
from jax.experimental.pallas import tpu_sc as plsc

N = 1 << 24
H = 1 << 23
NC = 2
NS = 16
NW = NC * NS
L = 16

SC_NUM, SC_DEN = 3, 4
S1 = H * SC_NUM // SC_DEN
W1 = S1 // NS
CH1 = 32768
NCH1 = W1 // CH1
UN1 = 8
TCB = 131072
TC_NBLK = (H - S1) // TCB

OPW = H // NW
CH2 = 32768
NCH2 = OPW // CH2
UN2 = 8

_mesh = plsc.VectorSubcoreMesh(
    core_axis_name="c", subcore_axis_name="s", num_cores=NC, num_subcores=NS
)


def _sums_body(psi_hbm, out_hbm, b0, b1, b2, accv, s0, s1, s2):
    wid = lax.axis_index("s") * NC + lax.axis_index("c")
    base = jnp.where(wid < NS, wid * W1, H + (wid - NS) * W1)
    bufs = (b0, b1, b2)
    sems = (s0, s1, s2)
    handles = [None, None, None]
    handles[0] = pltpu.async_copy(psi_hbm.at[pl.ds(base, CH1)], b0, s0)
    handles[1] = pltpu.async_copy(psi_hbm.at[pl.ds(base + CH1, CH1)], b1, s1)
    accs = (jnp.zeros((L,), jnp.float32),) * 4
    for g in range(NCH1):
        p = g % 3
        if g + 2 < NCH1:
            q = (g + 2) % 3
            handles[q] = pltpu.async_copy(
                psi_hbm.at[pl.ds(base + (g + 2) * CH1, CH1)], bufs[q],
                sems[q])
        handles[p].wait()
        buf = bufs[p]

        def body(j, a):
            a0, a1, a2, a3 = a
            off = j * (UN1 * L)
            xs = [buf[pl.ds(off + k * L, L)] for k in range(UN1)]
            a0 = a0 + xs[0] * xs[0]
            a1 = a1 + xs[1] * xs[1]
            a2 = a2 + xs[2] * xs[2]
            a3 = a3 + xs[3] * xs[3]
            a0 = a0 + xs[4] * xs[4]
            a1 = a1 + xs[5] * xs[5]
            a2 = a2 + xs[6] * xs[6]
            a3 = a3 + xs[7] * xs[7]
            return (a0, a1, a2, a3)

        accs = lax.fori_loop(0, CH1 // (UN1 * L), body, accs)
    accv[...] = (accs[0] + accs[1]) + (accs[2] + accs[3])
    pltpu.sync_copy(accv, out_hbm.at[wid])


_sums = pl.kernel(
    _sums_body,
    out_type=jax.ShapeDtypeStruct((NW, L), jnp.float32),
    mesh=_mesh,
    scratch_types=[
        pltpu.VMEM((CH1,), jnp.float32),
        pltpu.VMEM((CH1,), jnp.float32),
        pltpu.VMEM((CH1,), jnp.float32),
        pltpu.VMEM((L,), jnp.float32),
        pltpu.SemaphoreType.DMA,
        pltpu.SemaphoreType.DMA,
        pltpu.SemaphoreType.DMA,
    ],
)


def _make_copy(half_base):
    def _copy_body(psi_hbm, scale_hbm, out_hbm,
                   b0, b1, b2, scalev, s0, s1, s2, o0, o1, o2):
        wid = lax.axis_index("s") * NC + lax.axis_index("c")
        pltpu.sync_copy(scale_hbm, scalev)
        src0 = half_base + wid * OPW
        dst0 = wid * OPW
        sv = scalev[...]
        NB = 3
        bufs = (b0, b1, b2)
        gsems = (s0, s1, s2)
        osems = (o0, o1, o2)
        gh = [None] * NB
        oh = [None] * NB

        def gather(g):
            q = g % NB
            gh[q] = pltpu.async_copy(
                psi_hbm.at[pl.ds(src0 + g * CH2, CH2)], bufs[q], gsems[q])

        for g in range(min(NB - 1, NCH2)):
            gather(g)
        for g in range(NCH2):
            p = g % NB
            nxt = g + NB - 1
            if nxt < NCH2:
                q = nxt % NB
                if oh[q] is not None:
                    oh[q].wait()
                    oh[q] = None
                gather(nxt)
            gh[p].wait()
            buf = bufs[p]

            def mbody(j, _):
                off = j * (UN2 * L)
                for k in range(UN2):
                    buf[pl.ds(off + k * L, L)] = buf[pl.ds(off + k * L, L)] * sv
                return 0

            lax.fori_loop(0, CH2 // (UN2 * L), mbody, 0)
            oh[p] = pltpu.async_copy(
                buf, out_hbm.at[pl.ds(dst0 + g * CH2, CH2)], osems[p])
        for q in range(NB):
            if oh[q] is not None:
                oh[q].wait()

    return pl.kernel(
        _copy_body,
        out_type=jax.ShapeDtypeStruct((H,), jnp.float32),
        mesh=_mesh,
        scratch_types=[
            pltpu.VMEM((CH2,), jnp.float32),
            pltpu.VMEM((CH2,), jnp.float32),
            pltpu.VMEM((CH2,), jnp.float32),
            pltpu.VMEM((L,), jnp.float32),
            pltpu.SemaphoreType.DMA,
            pltpu.SemaphoreType.DMA,
            pltpu.SemaphoreType.DMA,
            pltpu.SemaphoreType.DMA,
            pltpu.SemaphoreType.DMA,
            pltpu.SemaphoreType.DMA,
        ],
    )


_copy0 = _make_copy(0)
_copy1 = _make_copy(H)


def _tc_tails_body(x_ref, o_ref):
    i = pl.program_id(0)
    j = pl.program_id(1)

    @pl.when(j == 0)
    def _():
        o_ref[i, 0] = 0.0

    x = x_ref[...]
    o_ref[i, 0] += jnp.sum(x * x)


_tc_tails = pl.pallas_call(
    _tc_tails_body,
    grid=(2, TC_NBLK),
    in_specs=[pl.BlockSpec((TCB,),
                           lambda i, j: i * (H // TCB) + S1 // TCB + j)],
    out_specs=pl.BlockSpec((2, 1), lambda i, j: (0, 0),
                           memory_space=pltpu.SMEM),
    out_shape=jax.ShapeDtypeStruct((2, 1), jnp.float32),
)


def kernel(psi, u):
    partials = _sums(psi)
    tails = _tc_tails(psi)
    mass0 = jnp.sum(partials[: NW // 2]) + tails[0, 0]
    total = mass0 + jnp.sum(partials[NW // 2:]) + tails[1, 0]
    p0 = mass0 / total
    outcome = u[0] > p0
    p_out = jnp.where(outcome, 1.0 - p0, p0)
    scale = 1.0 / jnp.sqrt(p_out)
    scale_arr = jnp.full((L,), scale, dtype=jnp.float32)
    return lax.cond(outcome,
                    lambda: _copy1(psi, scale_arr),
                    lambda: _copy0(psi, scale_arr))


# speedup vs baseline: 1.2046x; 1.2046x over previous
import jax
import jax.numpy as jnp
from jax import lax
from jax.experimental import pallas as pl
from jax.experimental.pallas import tpu as pltpu
from jax.experimental.pallas import tpu_sc as plsc

N = 1 << 24
H = 1 << 23
NC = 2
NS = 16
NW = NC * NS
L = 16

SC_NUM, SC_DEN = 3, 4
S1 = H * SC_NUM // SC_DEN
W1 = S1 // NS
CH1 = 32768
NCH1 = W1 // CH1
UN1 = 8
TCB = 131072
TC_NBLK = (H - S1) // TCB

OPW = H // NW
CH2 = 32768
NCH2 = OPW // CH2
UN2 = 8

_mesh = plsc.VectorSubcoreMesh(
    core_axis_name="c", subcore_axis_name="s", num_cores=NC, num_subcores=NS
)


def _sums_body(psi_hbm, out_hbm, b0, b1, b2, accv, s0, s1, s2):
    wid = lax.axis_index("s") * NC + lax.axis_index("c")
    base = jnp.where(wid < NS, wid * W1, H + (wid - NS) * W1)
    bufs = (b0, b1, b2)
    sems = (s0, s1, s2)
    handles = [None, None, None]
    handles[0] = pltpu.async_copy(psi_hbm.at[pl.ds(base, CH1)], b0, s0)
    handles[1] = pltpu.async_copy(psi_hbm.at[pl.ds(base + CH1, CH1)], b1, s1)
    accs = (jnp.zeros((L,), jnp.float32),) * 4
    for g in range(NCH1):
        p = g % 3
        if g + 2 < NCH1:
            q = (g + 2) % 3
            handles[q] = pltpu.async_copy(
                psi_hbm.at[pl.ds(base + (g + 2) * CH1, CH1)], bufs[q],
                sems[q])
        handles[p].wait()
        buf = bufs[p]

        def body(j, a):
            a0, a1, a2, a3 = a
            off = j * (UN1 * L)
            xs = [buf[pl.ds(off + k * L, L)] for k in range(UN1)]
            a0 = a0 + xs[0] * xs[0]
            a1 = a1 + xs[1] * xs[1]
            a2 = a2 + xs[2] * xs[2]
            a3 = a3 + xs[3] * xs[3]
            a0 = a0 + xs[4] * xs[4]
            a1 = a1 + xs[5] * xs[5]
            a2 = a2 + xs[6] * xs[6]
            a3 = a3 + xs[7] * xs[7]
            return (a0, a1, a2, a3)

        accs = lax.fori_loop(0, CH1 // (UN1 * L), body, accs)
    accv[...] = (accs[0] + accs[1]) + (accs[2] + accs[3])
    pltpu.sync_copy(accv, out_hbm.at[wid])


_sums = pl.kernel(
    _sums_body,
    out_type=jax.ShapeDtypeStruct((NW, L), jnp.float32),
    mesh=_mesh,
    scratch_types=[
        pltpu.VMEM((CH1,), jnp.float32),
        pltpu.VMEM((CH1,), jnp.float32),
        pltpu.VMEM((CH1,), jnp.float32),
        pltpu.VMEM((L,), jnp.float32),
        pltpu.SemaphoreType.DMA,
        pltpu.SemaphoreType.DMA,
        pltpu.SemaphoreType.DMA,
    ],
)


def _make_copy(half_base):
    def _copy_body(psi_hbm, scale_hbm, out_hbm,
                   b0, b1, b2, scalev, s0, s1, s2, o0, o1, o2):
        wid = lax.axis_index("s") * NC + lax.axis_index("c")
        pltpu.sync_copy(scale_hbm, scalev)
        src0 = half_base + wid * OPW
        dst0 = wid * OPW
        sv = scalev[...]
        NB = 3
        bufs = (b0, b1, b2)
        gsems = (s0, s1, s2)
        osems = (o0, o1, o2)
        gh = [None] * NB
        oh = [None] * NB

        def gather(g):
            q = g % NB
            gh[q] = pltpu.async_copy(
                psi_hbm.at[pl.ds(src0 + g * CH2, CH2)], bufs[q], gsems[q])

        for g in range(min(NB - 1, NCH2)):
            gather(g)
        for g in range(NCH2):
            p = g % NB
            nxt = g + NB - 1
            if nxt < NCH2:
                q = nxt % NB
                if oh[q] is not None:
                    oh[q].wait()
                    oh[q] = None
                gather(nxt)
            gh[p].wait()
            buf = bufs[p]

            def mbody(j, _):
                off = j * (UN2 * L)
                for k in range(UN2):
                    buf[pl.ds(off + k * L, L)] = buf[pl.ds(off + k * L, L)] * sv
                return 0

            lax.fori_loop(0, CH2 // (UN2 * L), mbody, 0)
            oh[p] = pltpu.async_copy(
                buf, out_hbm.at[pl.ds(dst0 + g * CH2, CH2)], osems[p])
        for q in range(NB):
            if oh[q] is not None:
                oh[q].wait()

    return pl.kernel(
        _copy_body,
        out_type=jax.ShapeDtypeStruct((H,), jnp.float32),
        mesh=_mesh,
        scratch_types=[
            pltpu.VMEM((CH2,), jnp.float32),
            pltpu.VMEM((CH2,), jnp.float32),
            pltpu.VMEM((CH2,), jnp.float32),
            pltpu.VMEM((L,), jnp.float32),
            pltpu.SemaphoreType.DMA,
            pltpu.SemaphoreType.DMA,
            pltpu.SemaphoreType.DMA,
            pltpu.SemaphoreType.DMA,
            pltpu.SemaphoreType.DMA,
            pltpu.SemaphoreType.DMA,
        ],
    )


_copy0 = _make_copy(0)
_copy1 = _make_copy(H)


def _tc_tails_body(x_ref, o_ref, acc):
    i = pl.program_id(0)
    j = pl.program_id(1)

    @pl.when(j == 0)
    def _():
        acc[...] = jnp.zeros_like(acc)

    x = x_ref[...].reshape(TCB // 1024, 8, 128)
    acc[...] += jnp.sum(x * x, axis=0)

    @pl.when(j == TC_NBLK - 1)
    def _():
        o_ref[i, 0] = jnp.sum(acc[...])


_tc_tails = pl.pallas_call(
    _tc_tails_body,
    grid=(2, TC_NBLK),
    in_specs=[pl.BlockSpec((TCB,),
                           lambda i, j: i * (H // TCB) + S1 // TCB + j)],
    out_specs=pl.BlockSpec((2, 1), lambda i, j: (0, 0),
                           memory_space=pltpu.SMEM),
    out_shape=jax.ShapeDtypeStruct((2, 1), jnp.float32),
    scratch_shapes=[pltpu.VMEM((8, 128), jnp.float32)],
)


def kernel(psi, u):
    partials = _sums(psi)
    tails = _tc_tails(psi)
    mass0 = jnp.sum(partials[: NW // 2]) + tails[0, 0]
    total = mass0 + jnp.sum(partials[NW // 2:]) + tails[1, 0]
    p0 = mass0 / total
    outcome = u[0] > p0
    p_out = jnp.where(outcome, 1.0 - p0, p0)
    scale = 1.0 / jnp.sqrt(p_out)
    scale_arr = jnp.full((L,), scale, dtype=jnp.float32)
    return lax.cond(outcome,
                    lambda: _copy1(psi, scale_arr),
                    lambda: _copy0(psi, scale_arr))


# speedup vs baseline: 1.2484x; 1.0364x over previous
import jax
import jax.numpy as jnp
from jax import lax
from jax.experimental import pallas as pl
from jax.experimental.pallas import tpu as pltpu
from jax.experimental.pallas import tpu_sc as plsc

N = 1 << 24
H = 1 << 23
NC = 2
NS = 16
NW = NC * NS
L = 16

SC_NUM, SC_DEN = 3, 4
S1 = H * SC_NUM // SC_DEN
W1 = S1 // NS
CH1 = 32768
NCH1 = W1 // CH1
UN1 = 8
TCB = 131072
TC_NBLK = (H - S1) // TCB

GW = 128
NROWS = N // GW
HROWS = NROWS // 2
RPW = HROWS // NW
CR = 128
NCH2 = RPW // CR

_mesh = plsc.VectorSubcoreMesh(
    core_axis_name="c", subcore_axis_name="s", num_cores=NC, num_subcores=NS
)


def _sums_body(psi_hbm, out_hbm, b0, b1, b2, accv, shv, s0, s1, s2):
    wid = lax.axis_index("s") * NC + lax.axis_index("c")
    base = jnp.where(wid < NS, wid * W1, H + (wid - NS) * W1)
    bufs = (b0, b1, b2)
    sems = (s0, s1, s2)
    handles = [None, None, None]
    handles[0] = pltpu.async_copy(psi_hbm.at[pl.ds(base, CH1)], b0, s0)
    handles[1] = pltpu.async_copy(psi_hbm.at[pl.ds(base + CH1, CH1)], b1, s1)
    accs = (jnp.zeros((L,), jnp.float32),) * 4
    for g in range(NCH1):
        p = g % 3
        if g + 2 < NCH1:
            q = (g + 2) % 3
            handles[q] = pltpu.async_copy(
                psi_hbm.at[pl.ds(base + (g + 2) * CH1, CH1)], bufs[q],
                sems[q])
        handles[p].wait()
        buf = bufs[p]

        def body(j, a):
            a0, a1, a2, a3 = a
            off = j * (UN1 * L)
            xs = [buf[pl.ds(off + k * L, L)] for k in range(UN1)]
            a0 = a0 + xs[0] * xs[0]
            a1 = a1 + xs[1] * xs[1]
            a2 = a2 + xs[2] * xs[2]
            a3 = a3 + xs[3] * xs[3]
            a0 = a0 + xs[4] * xs[4]
            a1 = a1 + xs[5] * xs[5]
            a2 = a2 + xs[6] * xs[6]
            a3 = a3 + xs[7] * xs[7]
            return (a0, a1, a2, a3)

        accs = lax.fori_loop(0, CH1 // (UN1 * L), body, accs)
    x = (accs[0] + accs[1]) + (accs[2] + accs[3])
    sid = lax.axis_index("s")
    accv[...] = jnp.zeros((L,), jnp.float32)
    pltpu.sync_copy(accv, shv.at[pl.ds(sid * L, L)])
    accv[...] = x
    idx = jnp.full((L,), sid * L, jnp.int32)
    pltpu.sync_copy(accv, shv.at[idx], add=True)
    pltpu.sync_copy(shv.at[idx], accv)
    pltpu.sync_copy(accv, out_hbm.at[wid])


_sums = pl.kernel(
    _sums_body,
    out_type=jax.ShapeDtypeStruct((NW, L), jnp.float32),
    mesh=_mesh,
    scratch_types=[
        pltpu.VMEM((CH1,), jnp.float32),
        pltpu.VMEM((CH1,), jnp.float32),
        pltpu.VMEM((CH1,), jnp.float32),
        pltpu.VMEM((L,), jnp.float32),
        pltpu.VMEM_SHARED((NS * L,), jnp.float32),
        pltpu.SemaphoreType.DMA,
        pltpu.SemaphoreType.DMA,
        pltpu.SemaphoreType.DMA,
    ],
)


def _tc_tails_body(x_ref, o_ref, acc):
    i = pl.program_id(0)
    j = pl.program_id(1)

    @pl.when(j == 0)
    def _():
        acc[...] = jnp.zeros_like(acc)

    x = x_ref[...].reshape(TCB // 1024, 8, 128)
    acc[...] += jnp.sum(x * x, axis=0)

    @pl.when(j == TC_NBLK - 1)
    def _():
        o_ref[i, 0] = jnp.sum(acc[...])


_tc_tails = pl.pallas_call(
    _tc_tails_body,
    grid=(2, TC_NBLK),
    in_specs=[pl.BlockSpec((TCB,),
                           lambda i, j: i * (H // TCB) + S1 // TCB + j)],
    out_specs=pl.BlockSpec((2, 1), lambda i, j: (0, 0),
                           memory_space=pltpu.SMEM),
    out_shape=jax.ShapeDtypeStruct((2, 1), jnp.float32),
    scratch_shapes=[pltpu.VMEM((8, 128), jnp.float32)],
)


def _copy_body(psi_hbm, part_hbm, aux_hbm, out_hbm,
               b0, b1, b2, pv, av, idxv, s0, s1, s2, o0, o1, o2):
    wid = lax.axis_index("s") * NC + lax.axis_index("c")
    pltpu.sync_copy(part_hbm, pv)
    pltpu.sync_copy(aux_hbm, av)

    acc0 = pv[0, :]
    for i in range(1, NW // 2):
        acc0 = acc0 + pv[i, :]
    acc1 = pv[NW // 2, :]
    for i in range(NW // 2 + 1, NW):
        acc1 = acc1 + pv[i, :]
    mass0 = acc0 + av[0, :]
    total = mass0 + acc1 + av[1, :]
    uvec = av[2, :]
    p0 = mass0 / total
    outcome = uvec > p0
    p_out = jnp.where(outcome, 1.0 - p0, p0)

    seed_i = 0x5F3759DF - lax.shift_right_logical(
        lax.bitcast_convert_type(p_out, jnp.int32), 1)
    y = lax.bitcast_convert_type(seed_i, jnp.float32)
    half = p_out * 0.5
    for _ in range(3):
        y = y * (1.5 - half * y * y)
    sv = y

    row0 = jnp.where(outcome, HROWS, 0) + wid * RPW
    iot = lax.iota(jnp.int32, L)

    def ibody(j, _):
        idxv[pl.ds(j * L, L)] = row0 + j * L + iot
        return 0

    lax.fori_loop(0, RPW // L, ibody, 0)

    dst0 = wid * RPW
    NB = 3
    bufs = (b0, b1, b2)
    gsems = (s0, s1, s2)
    osems = (o0, o1, o2)
    gh = [None] * NB
    oh = [None] * NB

    def gather(g):
        q = g % NB
        gh[q] = pltpu.async_copy(
            psi_hbm.at[idxv.at[pl.ds(g * CR, CR)]], bufs[q], gsems[q])

    for g in range(min(NB - 1, NCH2)):
        gather(g)
    for g in range(NCH2):
        p = g % NB
        nxt = g + NB - 1
        if nxt < NCH2:
            q = nxt % NB
            if oh[q] is not None:
                oh[q].wait()
                oh[q] = None
            gather(nxt)
        gh[p].wait()
        buf = bufs[p]

        def mbody(r, _):
            for k in range(GW // L):
                buf[r, pl.ds(k * L, L)] = buf[r, pl.ds(k * L, L)] * sv
            return 0

        lax.fori_loop(0, CR, mbody, 0)
        oh[p] = pltpu.async_copy(
            buf, out_hbm.at[pl.ds(dst0 + g * CR, CR)], osems[p])
    for q in range(NB):
        if oh[q] is not None:
            oh[q].wait()


_copy = pl.kernel(
    _copy_body,
    out_type=jax.ShapeDtypeStruct((HROWS, GW), jnp.float32),
    mesh=_mesh,
    scratch_types=[
        pltpu.VMEM((CR, GW), jnp.float32),
        pltpu.VMEM((CR, GW), jnp.float32),
        pltpu.VMEM((CR, GW), jnp.float32),
        pltpu.VMEM((NW, L), jnp.float32),
        pltpu.VMEM((3, L), jnp.float32),
        pltpu.VMEM((RPW,), jnp.int32),
        pltpu.SemaphoreType.DMA,
        pltpu.SemaphoreType.DMA,
        pltpu.SemaphoreType.DMA,
        pltpu.SemaphoreType.DMA,
        pltpu.SemaphoreType.DMA,
        pltpu.SemaphoreType.DMA,
    ],
)


def kernel(psi, u):
    partials = _sums(psi)
    tails = _tc_tails(psi)
    aux = jnp.stack([
        jnp.full((L,), tails[0, 0], dtype=jnp.float32),
        jnp.full((L,), tails[1, 0], dtype=jnp.float32),
        jnp.broadcast_to(u, (L,)).astype(jnp.float32),
    ])
    out2 = _copy(psi.reshape(NROWS, GW), partials, aux)
    return out2.reshape(H)


# speedup vs baseline: 1.2531x; 1.0037x over previous
import jax
import jax.numpy as jnp
from jax import lax
from jax.experimental import pallas as pl
from jax.experimental.pallas import tpu as pltpu
from jax.experimental.pallas import tpu_sc as plsc

N = 1 << 24
H = 1 << 23
NC = 2
NS = 16
NW = NC * NS
L = 16

SC_NUM, SC_DEN = 13, 16
S1 = H * SC_NUM // SC_DEN
W1 = S1 // NS
CH1 = 32768
NCH1 = W1 // CH1
UN1 = 8
TCB = 131072
TC_NBLK = (H - S1) // TCB

GW = 128
NROWS = N // GW
HROWS = NROWS // 2
RPW = HROWS // NW
CR = 128
NCH2 = RPW // CR

_mesh = plsc.VectorSubcoreMesh(
    core_axis_name="c", subcore_axis_name="s", num_cores=NC, num_subcores=NS
)


def _sums_body(psi_hbm, out_hbm, b0, b1, b2, accv, shv, s0, s1, s2):
    wid = lax.axis_index("s") * NC + lax.axis_index("c")
    base = jnp.where(wid < NS, wid * W1, H + (wid - NS) * W1)
    bufs = (b0, b1, b2)
    sems = (s0, s1, s2)
    handles = [None, None, None]
    handles[0] = pltpu.async_copy(psi_hbm.at[pl.ds(base, CH1)], b0, s0)
    handles[1] = pltpu.async_copy(psi_hbm.at[pl.ds(base + CH1, CH1)], b1, s1)
    accs = (jnp.zeros((L,), jnp.float32),) * 4
    for g in range(NCH1):
        p = g % 3
        if g + 2 < NCH1:
            q = (g + 2) % 3
            handles[q] = pltpu.async_copy(
                psi_hbm.at[pl.ds(base + (g + 2) * CH1, CH1)], bufs[q],
                sems[q])
        handles[p].wait()
        buf = bufs[p]

        def body(j, a):
            a0, a1, a2, a3 = a
            off = j * (UN1 * L)
            xs = [buf[pl.ds(off + k * L, L)] for k in range(UN1)]
            a0 = a0 + xs[0] * xs[0]
            a1 = a1 + xs[1] * xs[1]
            a2 = a2 + xs[2] * xs[2]
            a3 = a3 + xs[3] * xs[3]
            a0 = a0 + xs[4] * xs[4]
            a1 = a1 + xs[5] * xs[5]
            a2 = a2 + xs[6] * xs[6]
            a3 = a3 + xs[7] * xs[7]
            return (a0, a1, a2, a3)

        accs = lax.fori_loop(0, CH1 // (UN1 * L), body, accs)
    x = (accs[0] + accs[1]) + (accs[2] + accs[3])
    sid = lax.axis_index("s")
    accv[...] = jnp.zeros((L,), jnp.float32)
    pltpu.sync_copy(accv, shv.at[pl.ds(sid * L, L)])
    accv[...] = x
    idx = jnp.full((L,), sid * L, jnp.int32)
    pltpu.sync_copy(accv, shv.at[idx], add=True)
    pltpu.sync_copy(shv.at[idx], accv)
    pltpu.sync_copy(accv, out_hbm.at[wid])


_sums = pl.kernel(
    _sums_body,
    out_type=jax.ShapeDtypeStruct((NW, L), jnp.float32),
    mesh=_mesh,
    scratch_types=[
        pltpu.VMEM((CH1,), jnp.float32),
        pltpu.VMEM((CH1,), jnp.float32),
        pltpu.VMEM((CH1,), jnp.float32),
        pltpu.VMEM((L,), jnp.float32),
        pltpu.VMEM_SHARED((NS * L,), jnp.float32),
        pltpu.SemaphoreType.DMA,
        pltpu.SemaphoreType.DMA,
        pltpu.SemaphoreType.DMA,
    ],
)


def _tc_tails_body(x_ref, o_ref, acc):
    i = pl.program_id(0)
    j = pl.program_id(1)

    @pl.when(j == 0)
    def _():
        acc[...] = jnp.zeros_like(acc)

    x = x_ref[...].reshape(TCB // 1024, 8, 128)
    acc[...] += jnp.sum(x * x, axis=0)

    @pl.when(j == TC_NBLK - 1)
    def _():
        o_ref[i, 0] = jnp.sum(acc[...])


_tc_tails = pl.pallas_call(
    _tc_tails_body,
    grid=(2, TC_NBLK),
    in_specs=[pl.BlockSpec((TCB,),
                           lambda i, j: i * (H // TCB) + S1 // TCB + j)],
    out_specs=pl.BlockSpec((2, 1), lambda i, j: (0, 0),
                           memory_space=pltpu.SMEM),
    out_shape=jax.ShapeDtypeStruct((2, 1), jnp.float32),
    scratch_shapes=[pltpu.VMEM((8, 128), jnp.float32)],
)


def _copy_body(psi_hbm, part_hbm, aux_hbm, out_hbm,
               b0, b1, b2, pv, av, idxv, s0, s1, s2, o0, o1, o2):
    wid = lax.axis_index("s") * NC + lax.axis_index("c")
    pltpu.sync_copy(part_hbm, pv)
    pltpu.sync_copy(aux_hbm, av)

    acc0 = pv[0, :]
    for i in range(1, NW // 2):
        acc0 = acc0 + pv[i, :]
    acc1 = pv[NW // 2, :]
    for i in range(NW // 2 + 1, NW):
        acc1 = acc1 + pv[i, :]
    mass0 = acc0 + av[0, :]
    total = mass0 + acc1 + av[1, :]
    uvec = av[2, :]
    p0 = mass0 / total
    outcome = uvec > p0
    p_out = jnp.where(outcome, 1.0 - p0, p0)

    seed_i = 0x5F3759DF - lax.shift_right_logical(
        lax.bitcast_convert_type(p_out, jnp.int32), 1)
    y = lax.bitcast_convert_type(seed_i, jnp.float32)
    half = p_out * 0.5
    for _ in range(3):
        y = y * (1.5 - half * y * y)
    sv = y

    row0 = jnp.where(outcome, HROWS, 0) + wid * RPW
    iot = lax.iota(jnp.int32, L)

    def build_idx(g):
        q = g % NB

        def ibody(j, _):
            idxv[pl.ds(q * CR + j * L, L)] = row0 + g * CR + j * L + iot
            return 0

        lax.fori_loop(0, CR // L, ibody, 0)

    dst0 = wid * RPW
    NB = 3
    bufs = (b0, b1, b2)
    gsems = (s0, s1, s2)
    osems = (o0, o1, o2)
    gh = [None] * NB
    oh = [None] * NB

    def gather(g):
        q = g % NB
        build_idx(g)
        gh[q] = pltpu.async_copy(
            psi_hbm.at[idxv.at[pl.ds(q * CR, CR)]], bufs[q], gsems[q])

    for g in range(min(NB - 1, NCH2)):
        gather(g)
    for g in range(NCH2):
        p = g % NB
        nxt = g + NB - 1
        if nxt < NCH2:
            q = nxt % NB
            if oh[q] is not None:
                oh[q].wait()
                oh[q] = None
            gather(nxt)
        gh[p].wait()
        buf = bufs[p]

        def mbody(r, _):
            for k in range(GW // L):
                buf[r, pl.ds(k * L, L)] = buf[r, pl.ds(k * L, L)] * sv
            return 0

        lax.fori_loop(0, CR, mbody, 0)
        oh[p] = pltpu.async_copy(
            buf, out_hbm.at[pl.ds(dst0 + g * CR, CR)], osems[p])
    for q in range(NB):
        if oh[q] is not None:
            oh[q].wait()


_copy = pl.kernel(
    _copy_body,
    out_type=jax.ShapeDtypeStruct((HROWS, GW), jnp.float32),
    mesh=_mesh,
    scratch_types=[
        pltpu.VMEM((CR, GW), jnp.float32),
        pltpu.VMEM((CR, GW), jnp.float32),
        pltpu.VMEM((CR, GW), jnp.float32),
        pltpu.VMEM((NW, L), jnp.float32),
        pltpu.VMEM((3, L), jnp.float32),
        pltpu.VMEM((3 * CR,), jnp.int32),
        pltpu.SemaphoreType.DMA,
        pltpu.SemaphoreType.DMA,
        pltpu.SemaphoreType.DMA,
        pltpu.SemaphoreType.DMA,
        pltpu.SemaphoreType.DMA,
        pltpu.SemaphoreType.DMA,
    ],
)


def kernel(psi, u):
    partials = _sums(psi)
    tails = _tc_tails(psi)
    aux = jnp.stack([
        jnp.full((L,), tails[0, 0], dtype=jnp.float32),
        jnp.full((L,), tails[1, 0], dtype=jnp.float32),
        jnp.broadcast_to(u, (L,)).astype(jnp.float32),
    ])
    out2 = _copy(psi.reshape(NROWS, GW), partials, aux)
    return out2.reshape(H)
